# Initial kernel scaffold; baseline (speedup 1.0000x reference)
import jax
import jax.numpy as jnp
from jax.experimental import pallas as pl


def kernel(x, edge_index, batch, u, Wl1, Wr1, att1, b1, Wl2, Wr2, att2, b2, W_lin1, b_lin1, W_out, b_out):
    raise NotImplementedError("write your pallas kernel here")


# speedup vs baseline: 1.0005x; 1.0005x over previous
import jax
import jax.numpy as jnp
from jax.experimental import pallas as pl

_N = 10000
_G = 32
_H = 8
_C = 32


def _relu_pallas(x):
    def body(x_ref, o_ref):
        o_ref[...] = jnp.maximum(x_ref[...], 0.0)

    return pl.pallas_call(
        body, out_shape=jax.ShapeDtypeStruct(x.shape, x.dtype)
    )(x)


def _gatv2(x, src, dst, Wl, Wr, att, b, n):
    xl = (x @ Wl).reshape(-1, _H, _C)
    xr = (x @ Wr).reshape(-1, _H, _C)
    m = jax.nn.leaky_relu(xl[src] + xr[dst], 0.2)
    e = (m * att[None]).sum(-1)
    emax = jax.ops.segment_max(e, dst, num_segments=n)
    ex = jnp.exp(e - emax[dst])
    den = jax.ops.segment_sum(ex, dst, num_segments=n)
    alpha = ex / (den[dst] + 1e-16)
    out = jax.ops.segment_sum(xl[src] * alpha[:, :, None], dst, num_segments=n)
    return out.reshape(n, _H * _C) + b


def kernel(x, edge_index, batch, u, Wl1, Wr1, att1, b1, Wl2, Wr2, att2, b2,
           W_lin1, b_lin1, W_out, b_out):
    loop = jnp.arange(_N, dtype=edge_index.dtype)
    src = jnp.concatenate([edge_index[0], loop])
    dst = jnp.concatenate([edge_index[1], loop])
    h = _relu_pallas(_gatv2(x, src, dst, Wl1, Wr1, att1, b1, _N))
    h = _relu_pallas(_gatv2(h, src, dst, Wl2, Wr2, att2, b2, _N))
    cnt = jax.ops.segment_sum(jnp.ones((_N,), jnp.float32), batch, num_segments=_G)
    pooled = jax.ops.segment_sum(h, batch, num_segments=_G) / jnp.maximum(cnt, 1.0)[:, None]
    h = jax.nn.relu(pooled @ W_lin1 + b_lin1)
    h = jnp.concatenate([h, u], axis=1)
    return (h @ W_out + b_out).squeeze(1)


# speedup vs baseline: 9.0830x; 9.0786x over previous
import dataclasses

import jax
import jax.numpy as jnp
from jax import lax
from jax.experimental import pallas as pl
from jax.experimental.pallas import tpu as pltpu
from jax.experimental.pallas import tpu_sc as plsc

_N = 10000
_E = 320000
_G = 32
_H = 8
_C = 32
_F = 256
_FH = 128
_FQ = 64
_HL = 4

_NP = 10240
_EP = 331776
_K = 128
_EPT = _EP // 16
_NCH = _EPT // _K
_RT = _NP // 16
_RH = _RT // 2

_mesh = plsc.VectorSubcoreMesh(
    core_axis_name="c", subcore_axis_name="s", num_cores=2, num_subcores=16
)

_sc_cp = pltpu.CompilerParams(
    needs_layout_passes=False, use_tc_tiling_on_sc=False)


def _sc_gat_body(xlq, xrq, xle, srch, dsth, atth, bh, ho, exo,
                 dentab, rql0, rql1, rqr0, rqr1, rqb, srcc, dstc,
                 idxa, idxb, idxc, idxd, exs, als, attv, bv, fin, tmpa,
                 idx4, outacc, densh, sem1, sem2, sem3, sem4):
    cid = lax.axis_index("c")
    sid = lax.axis_index("s")
    base_t = sid * _EPT
    r0 = sid * _RT
    lanes = lax.iota(jnp.int32, 16)
    zero16 = jnp.zeros((16,), jnp.float32)
    qoff = [jnp.full((16,), (2 * cid + q) * _NP, jnp.int32) for q in (0, 1)]
    eoff = [jnp.full((16,), (4 * cid + h) * _NP, jnp.int32)
            for h in range(_HL)]

    pltpu.sync_copy(atth.at[pl.ds(cid * _HL, _HL)], attv)
    pltpu.sync_copy(bh.at[cid], bv)
    att_vecs = [[attv[h, pl.ds(0, 16)], attv[h, pl.ds(16, 16)]]
                for h in range(_HL)]

    plsc.store_scatter(idx4, [lanes], lanes, mask=lanes < 4)

    for h in range(_HL):
        @pl.loop(0, _NP, step=16)
        def _(j, h=h):
            dentab[h, pl.ds(j, 16)] = zero16

    @pl.loop(0, _RT, step=16)
    def _(j):
        for h in range(_HL):
            tmpa[h, pl.ds(j, 16)] = zero16
    pltpu.sync_copy(tmpa, densh.at[:, pl.ds(r0, _RT)])

    @pl.loop(0, _NCH)
    def _(ci):
        gb = base_t + ci * _K
        pltpu.sync_copy(srch.at[pl.ds(gb, _K)], srcc)
        pltpu.sync_copy(dsth.at[pl.ds(gb, _K)], dstc)
        for j in range(_K // 16):
            s = srcc[pl.ds(j * 16, 16)]
            d = dstc[pl.ds(j * 16, 16)]
            idxa[pl.ds(j * 16, 16)] = s + qoff[0]
            idxb[pl.ds(j * 16, 16)] = s + qoff[1]
            idxc[pl.ds(j * 16, 16)] = d + qoff[0]
            idxd[pl.ds(j * 16, 16)] = d + qoff[1]
        c1 = pltpu.async_copy(xlq.at[idxa], rql0, sem1)
        c2 = pltpu.async_copy(xlq.at[idxb], rql1, sem2)
        c3 = pltpu.async_copy(xrq.at[idxc], rqr0, sem3)
        c4 = pltpu.async_copy(xrq.at[idxd], rqr1, sem4)
        c1.wait()
        c2.wait()
        c3.wait()
        c4.wait()

        @pl.loop(0, _K, step=16)
        def _(e0):
            elanes = lanes + e0
            acc = [jnp.zeros((16,), jnp.float32) for _ in range(_HL)]
            for c in range(_FH):
                h = c // _C
                rl = rql0 if c < _FQ else rql1
                rr = rqr0 if c < _FQ else rqr1
                csp = jnp.full((16,), c % _FQ, jnp.int32)
                v = (plsc.load_gather(rl, [elanes, csp])
                     + plsc.load_gather(rr, [elanes, csp]))
                m = jnp.maximum(v, v * 0.2)
                acc[h] = acc[h] + m * att_vecs[h][(c % _C) // 16][c % 16]
            dl = dstc[pl.ds(e0, 16)]
            for h in range(_HL):
                hsp = jnp.full((16,), h, jnp.int32)
                ev = jnp.exp(acc[h])
                exs[h, pl.ds(e0, 16)] = ev
                plsc.addupdate_scatter(dentab, [hsp, dl], ev)
        pltpu.sync_copy(exs, exo.at[cid, :, pl.ds(gb, _K)])

    plsc.subcore_barrier()
    pltpu.sync_copy(dentab, densh.at[idx4], add=True)
    plsc.subcore_barrier()
    pltpu.sync_copy(densh.at[:, pl.ds(r0, _RT)], tmpa)
    @pl.loop(0, _RT, step=16)
    def _(j):
        for h in range(_HL):
            tmpa[h, pl.ds(j, 16)] = 1.0 / (tmpa[h, pl.ds(j, 16)] + 1e-30)
    pltpu.sync_copy(tmpa, densh.at[:, pl.ds(r0, _RT)])
    plsc.subcore_barrier()
    pltpu.sync_copy(densh, dentab)

    for rnd in range(_HL):
        @pl.loop(0, _RH)
        def _(r):
            for j in range(_C // 16):
                fin[r, pl.ds(j * 16, 16)] = zero16
        pltpu.sync_copy(fin, outacc.at[pl.ds(r0, _RH)])
        pltpu.sync_copy(fin, outacc.at[pl.ds(r0 + _RH, _RH)])
        plsc.subcore_barrier()

        @pl.loop(0, _NCH)
        def _(ci, rnd=rnd):
            gb = base_t + ci * _K
            pltpu.sync_copy(srch.at[pl.ds(gb, _K)], srcc)
            pltpu.sync_copy(dsth.at[pl.ds(gb, _K)], dstc)
            for j in range(_K // 16):
                idxa[pl.ds(j * 16, 16)] = (srcc[pl.ds(j * 16, 16)]
                                           + eoff[rnd])
            c1 = pltpu.async_copy(xle.at[idxa], rqb, sem1)
            pltpu.sync_copy(exo.at[cid, rnd, pl.ds(gb, _K)], exs.at[rnd])
            hsp = jnp.full((16,), rnd, jnp.int32)
            for j in range(_K // 16):
                r = plsc.load_gather(
                    dentab, [hsp, dstc[pl.ds(j * 16, 16)]])
                als[rnd, pl.ds(j * 16, 16)] = (
                    exs[rnd, pl.ds(j * 16, 16)] * r)
            c1.wait()

            @pl.loop(0, _K)
            def _(e, rnd=rnd):
                esp = jnp.full((16,), e, jnp.int32)
                hsp2 = jnp.full((16,), rnd, jnp.int32)
                a = plsc.load_gather(als, [hsp2, esp])
                rqb[e, pl.ds(0, 16)] = rqb[e, pl.ds(0, 16)] * a
                rqb[e, pl.ds(16, 16)] = rqb[e, pl.ds(16, 16)] * a

            pltpu.sync_copy(rqb, outacc.at[dstc], add=True)

        plsc.subcore_barrier()

        for half in range(2):
            pltpu.sync_copy(outacc.at[pl.ds(r0 + half * _RH, _RH)], fin)

            @pl.loop(0, _RH)
            def _(r, rnd=rnd):
                for j in range(_C // 16):
                    v = (fin[r, pl.ds(j * 16, 16)]
                         + bv[pl.ds(rnd * _C + j * 16, 16)])
                    fin[r, pl.ds(j * 16, 16)] = jnp.maximum(v, 0.0)
            pltpu.sync_copy(
                fin,
                ho.at[pl.ds((4 * cid + rnd) * _NP + r0 + half * _RH, _RH)])
        plsc.subcore_barrier()


_sc_gat = pl.kernel(
    _sc_gat_body,
    out_type=[
        jax.ShapeDtypeStruct((_H * _NP, _C), jnp.float32),
        jax.ShapeDtypeStruct((2, _HL, _EP), jnp.float32),
    ],
    mesh=_mesh,
    scratch_types=[
        pltpu.VMEM((_HL, _NP), jnp.float32),
        pltpu.VMEM((_K, _FQ), jnp.float32),
        pltpu.VMEM((_K, _FQ), jnp.float32),
        pltpu.VMEM((_K, _FQ), jnp.float32),
        pltpu.VMEM((_K, _FQ), jnp.float32),
        pltpu.VMEM((_K, _C), jnp.float32),
        pltpu.VMEM((_K,), jnp.int32),
        pltpu.VMEM((_K,), jnp.int32),
        pltpu.VMEM((_K,), jnp.int32),
        pltpu.VMEM((_K,), jnp.int32),
        pltpu.VMEM((_K,), jnp.int32),
        pltpu.VMEM((_K,), jnp.int32),
        pltpu.VMEM((_HL, _K), jnp.float32),
        pltpu.VMEM((_HL, _K), jnp.float32),
        pltpu.VMEM((_HL, _C), jnp.float32),
        pltpu.VMEM((_FH,), jnp.float32),
        pltpu.VMEM((_RH, _C), jnp.float32),
        pltpu.VMEM((_HL, _RT), jnp.float32),
        pltpu.VMEM((4,), jnp.int32),
        pltpu.VMEM_SHARED((_NP, _C), jnp.float32),
        pltpu.VMEM_SHARED((_HL, _NP), jnp.float32),
        pltpu.SemaphoreType.DMA,
        pltpu.SemaphoreType.DMA,
        pltpu.SemaphoreType.DMA,
        pltpu.SemaphoreType.DMA,
    ],
    compiler_params=_sc_cp,
)


_NB = 2048


def _write_layouts(xl, xr, xlo_ref, xro_ref, xle_ref):
    for q in range(4):
        xlo_ref[q] = xl[:, q * _FQ:(q + 1) * _FQ]
        xro_ref[q] = xr[:, q * _FQ:(q + 1) * _FQ]
    for g in range(_H):
        xle_ref[g] = xl[:, g * _C:(g + 1) * _C]


def _tc_lin1_body(x_ref, wl_ref, wr_ref, xlo_ref, xro_ref, xle_ref):
    x = x_ref[...]
    xl = jnp.dot(x, wl_ref[...], preferred_element_type=jnp.float32)
    xr = jnp.dot(x, wr_ref[...], preferred_element_type=jnp.float32)
    _write_layouts(xl, xr, xlo_ref, xro_ref, xle_ref)


def _tc_lin2_body(h_ref, wl_ref, wr_ref, xlo_ref, xro_ref, xle_ref):
    xl = sum(jnp.dot(h_ref[g], wl_ref[g * _C:(g + 1) * _C, :],
                     preferred_element_type=jnp.float32) for g in range(_H))
    xr = sum(jnp.dot(h_ref[g], wr_ref[g * _C:(g + 1) * _C, :],
                     preferred_element_type=jnp.float32) for g in range(_H))
    _write_layouts(xl, xr, xlo_ref, xro_ref, xle_ref)


def _lin_specs(first_in_spec, wshape):
    grid = (_NP // _NB,)
    in_specs = [
        first_in_spec,
        pl.BlockSpec(wshape, lambda i: (0, 0)),
        pl.BlockSpec(wshape, lambda i: (0, 0)),
    ]
    out_specs = [
        pl.BlockSpec((4, _NB, _FQ), lambda i: (0, i, 0)),
        pl.BlockSpec((4, _NB, _FQ), lambda i: (0, i, 0)),
        pl.BlockSpec((_H, _NB, _C), lambda i: (0, i, 0)),
    ]
    out_shape = [jax.ShapeDtypeStruct((4, _NP, _FQ), jnp.float32),
                 jax.ShapeDtypeStruct((4, _NP, _FQ), jnp.float32),
                 jax.ShapeDtypeStruct((_H, _NP, _C), jnp.float32)]
    return dict(grid=grid, in_specs=in_specs, out_specs=out_specs,
                out_shape=out_shape)


def _tc_head_body(h_ref, batch_ref, u_ref, wlin_ref, blin_ref,
                  wo1_ref, wo2_ref, out_ref):
    b = batch_ref[...]
    gids = lax.broadcasted_iota(jnp.int32, (_NP, _G), 1)
    oh = jnp.where(b == gids, 1.0, 0.0).astype(jnp.float32)
    cnt = jnp.sum(oh, axis=0)[:, None]
    pooled = jnp.concatenate(
        [lax.dot_general(oh, h_ref[g], (((0,), (0,)), ((), ())),
                         preferred_element_type=jnp.float32)
         for g in range(_H)], axis=1)
    pooled = pooled / jnp.maximum(cnt, 1.0)
    hr = jnp.maximum(
        jnp.dot(pooled, wlin_ref[...], preferred_element_type=jnp.float32)
        + blin_ref[...][None, :], 0.0)
    out = (jnp.dot(hr, wo1_ref[...], preferred_element_type=jnp.float32)
           + jnp.dot(u_ref[...], wo2_ref[...],
                     preferred_element_type=jnp.float32))
    out_ref[...] = out


def kernel(x, edge_index, batch, u, Wl1, Wr1, att1, b1, Wl2, Wr2, att2, b2,
           W_lin1, b_lin1, W_out, b_out):
    f32 = jnp.float32
    loop = jnp.arange(_N, dtype=edge_index.dtype)
    pad_e = jnp.full((_EP - _E - _N,), _N, edge_index.dtype)
    srcp = jnp.concatenate([edge_index[0], loop, pad_e])
    dstp = jnp.concatenate([edge_index[1], loop, pad_e])

    x_pad = jnp.pad(x, ((0, _NP - _N), (0, 5)))
    wl1p = jnp.pad(Wl1, ((0, 5), (0, 0)))
    wr1p = jnp.pad(Wr1, ((0, 5), (0, 0)))
    att1f = att1.reshape(_H, _C)
    att2f = att2.reshape(_H, _C)
    b1_2d = b1.reshape(2, _FH)
    b2_2d = b2.reshape(2, _FH)
    batch2d = jnp.concatenate(
        [batch, jnp.full((_NP - _N,), _G, batch.dtype)])[:, None]
    u_pad = jnp.pad(u, ((0, 0), (0, 5)))
    wlinp = jnp.pad(W_lin1, ((0, 0), (0, _FH - 64)))
    blinp = jnp.pad(b_lin1, (0, _FH - 64))
    wo1p = jnp.pad(W_out[:64], ((0, _FH - 64), (0, _FH - 1)))
    wo2p = jnp.pad(W_out[64:67], ((0, 5), (0, _FH - 1)))

    xl1q, xr1q, xl1e = pl.pallas_call(
        _tc_lin1_body,
        **_lin_specs(pl.BlockSpec((_NB, 8), lambda i: (i, 0)), (8, _F)),
    )(x_pad, wl1p, wr1p)
    h1e, _ = _sc_gat(xl1q.reshape(4 * _NP, _FQ),
                     xr1q.reshape(4 * _NP, _FQ),
                     xl1e.reshape(_H * _NP, _C),
                     srcp, dstp, att1f, b1_2d)

    xl2q, xr2q, xl2e = pl.pallas_call(
        _tc_lin2_body,
        **_lin_specs(pl.BlockSpec((_H, _NB, _C), lambda i: (0, i, 0)),
                     (_F, _F)),
    )(h1e.reshape(_H, _NP, _C), Wl2, Wr2)
    h2e, _ = _sc_gat(xl2q.reshape(4 * _NP, _FQ),
                     xr2q.reshape(4 * _NP, _FQ),
                     xl2e.reshape(_H * _NP, _C),
                     srcp, dstp, att2f, b2_2d)

    out = pl.pallas_call(
        _tc_head_body,
        out_shape=jax.ShapeDtypeStruct((_G, _FH), f32),
    )(h2e.reshape(_H, _NP, _C), batch2d, u_pad, wlinp, blinp, wo1p, wo2p)
    return out[:, 0] + b_out[0]


# speedup vs baseline: 9.4461x; 1.0400x over previous
import jax
import jax.numpy as jnp
from jax import lax
from jax.experimental import pallas as pl
from jax.experimental.pallas import tpu as pltpu
from jax.experimental.pallas import tpu_sc as plsc

_N = 10000
_E = 320000
_G = 32
_H = 8
_C = 32
_F = 256
_FH = 128
_HL = 4

_NP = 10240
_EP = 331776
_K = 128
_EPT = _EP // 16
_NCH = _EPT // _K
_RT = _NP // 16
_RH = _RT // 2

_mesh = plsc.VectorSubcoreMesh(
    core_axis_name="c", subcore_axis_name="s", num_cores=2, num_subcores=16
)

_sc_cp = pltpu.CompilerParams(
    needs_layout_passes=False, use_tc_tiling_on_sc=False)


def _sc_gat_body(xle, xre, srch, dsth, atth, bh, ho,
                 dentab, rxl, rxr, srcc, dstc, idxa, idxb,
                 evb, attv, bv, fin, dentr, tmpa, idx1,
                 outacc, densh, sem1, sem2):
    cid = lax.axis_index("c")
    sid = lax.axis_index("s")
    base_t = sid * _EPT
    r0 = sid * _RT
    lanes = lax.iota(jnp.int32, 16)
    zero16 = jnp.zeros((16,), jnp.float32)
    zsp = jnp.zeros((16,), jnp.int32)
    hoff = [jnp.full((16,), (4 * cid + h) * _NP, jnp.int32)
            for h in range(_HL)]

    pltpu.sync_copy(atth.at[pl.ds(cid * _HL, _HL)], attv)
    pltpu.sync_copy(bh.at[cid], bv)
    att_vecs = [[attv[h, pl.ds(0, 16)], attv[h, pl.ds(16, 16)]]
                for h in range(_HL)]

    for h in range(_HL):
        @pl.loop(0, _NP, step=16)
        def _(j, h=h):
            dentab[h, pl.ds(j, 16)] = zero16

    @pl.loop(0, _RT, step=16)
    def _(j):
        for h in range(_HL):
            tmpa[h, pl.ds(j, 16)] = zero16
    pltpu.sync_copy(tmpa, densh.at[:, pl.ds(r0, _RT)])

    for h in range(_HL):
        @pl.loop(0, _RH)
        def _(r):
            fin[r, pl.ds(0, 16)] = zero16
            fin[r, pl.ds(16, 16)] = zero16
        pltpu.sync_copy(fin, outacc.at[pl.ds(r0, _RH)])
        pltpu.sync_copy(fin, outacc.at[pl.ds(r0 + _RH, _RH)])
        plsc.subcore_barrier()

        @pl.loop(0, _NCH)
        def _(ci, h=h):
            gb = base_t + ci * _K
            pltpu.sync_copy(srch.at[pl.ds(gb, _K)], srcc)
            pltpu.sync_copy(dsth.at[pl.ds(gb, _K)], dstc)
            for j in range(_K // 16):
                idxa[pl.ds(j * 16, 16)] = srcc[pl.ds(j * 16, 16)] + hoff[h]
                idxb[pl.ds(j * 16, 16)] = dstc[pl.ds(j * 16, 16)] + hoff[h]
            c1 = pltpu.async_copy(xle.at[idxa], rxl, sem1)
            c2 = pltpu.async_copy(xre.at[idxb], rxr, sem2)
            c1.wait()
            c2.wait()

            hsp = jnp.full((16,), h, jnp.int32)

            @pl.loop(0, _K, step=16)
            def _(e0, h=h, hsp=hsp):
                elanes = lanes + e0
                acc = jnp.zeros((16,), jnp.float32)
                for c in range(_C):
                    csp = jnp.full((16,), c, jnp.int32)
                    v = (plsc.load_gather(rxl, [elanes, csp])
                         + plsc.load_gather(rxr, [elanes, csp]))
                    m = jnp.maximum(v, v * 0.2)
                    acc = acc + m * att_vecs[h][c // 16][c % 16]
                ev = jnp.exp(acc)
                evb[0, pl.ds(e0, 16)] = ev
                plsc.addupdate_scatter(
                    dentab, [hsp, dstc[pl.ds(e0, 16)]], ev)

            @pl.loop(0, _K)
            def _(e):
                esp = jnp.full((16,), e, jnp.int32)
                a = plsc.load_gather(evb, [zsp, esp])
                rxl[e, pl.ds(0, 16)] = rxl[e, pl.ds(0, 16)] * a
                rxl[e, pl.ds(16, 16)] = rxl[e, pl.ds(16, 16)] * a

            pltpu.sync_copy(rxl, outacc.at[dstc], add=True)

        plsc.subcore_barrier()
        plsc.store_scatter(idx1, [lanes],
                           jnp.full((16,), h, jnp.int32), mask=lanes < 1)
        pltpu.sync_copy(dentab.at[pl.ds(h, 1)], densh.at[idx1], add=True)
        plsc.subcore_barrier()

        pltpu.sync_copy(densh.at[pl.ds(h, 1), pl.ds(r0, _RT)], dentr)
        @pl.loop(0, _RT, step=16)
        def _(j):
            dentr[0, pl.ds(j, 16)] = 1.0 / (dentr[0, pl.ds(j, 16)] + 1e-30)

        for half in range(2):
            pltpu.sync_copy(outacc.at[pl.ds(r0 + half * _RH, _RH)], fin)

            @pl.loop(0, _RH)
            def _(r, h=h, half=half):
                rsp = jnp.full((16,), r + half * _RH, jnp.int32)
                a = plsc.load_gather(dentr, [zsp, rsp])
                for j in range(_C // 16):
                    v = (fin[r, pl.ds(j * 16, 16)] * a
                         + bv[pl.ds(h * _C + j * 16, 16)])
                    fin[r, pl.ds(j * 16, 16)] = jnp.maximum(v, 0.0)
            pltpu.sync_copy(
                fin,
                ho.at[pl.ds((4 * cid + h) * _NP + r0 + half * _RH, _RH)])
        plsc.subcore_barrier()


_sc_gat = pl.kernel(
    _sc_gat_body,
    out_type=jax.ShapeDtypeStruct((_H * _NP, _C), jnp.float32),
    mesh=_mesh,
    scratch_types=[
        pltpu.VMEM((_HL, _NP), jnp.float32),
        pltpu.VMEM((_K, _C), jnp.float32),
        pltpu.VMEM((_K, _C), jnp.float32),
        pltpu.VMEM((_K,), jnp.int32),
        pltpu.VMEM((_K,), jnp.int32),
        pltpu.VMEM((_K,), jnp.int32),
        pltpu.VMEM((_K,), jnp.int32),
        pltpu.VMEM((1, _K), jnp.float32),
        pltpu.VMEM((_HL, _C), jnp.float32),
        pltpu.VMEM((_FH,), jnp.float32),
        pltpu.VMEM((_RH, _C), jnp.float32),
        pltpu.VMEM((1, _RT), jnp.float32),
        pltpu.VMEM((_HL, _RT), jnp.float32),
        pltpu.VMEM((1,), jnp.int32),
        pltpu.VMEM_SHARED((_NP, _C), jnp.float32),
        pltpu.VMEM_SHARED((_HL, _NP), jnp.float32),
        pltpu.SemaphoreType.DMA,
        pltpu.SemaphoreType.DMA,
    ],
    compiler_params=_sc_cp,
)


_NB = 2048
_FQ = 64


def _write_eighths(xl, xr, xle_ref, xre_ref):
    for g in range(_H):
        xle_ref[g] = xl[:, g * _C:(g + 1) * _C]
        xre_ref[g] = xr[:, g * _C:(g + 1) * _C]


def _tc_lin1_body(x_ref, wl_ref, wr_ref, xle_ref, xre_ref):
    x = x_ref[...]
    xl = jnp.dot(x, wl_ref[...], preferred_element_type=jnp.float32)
    xr = jnp.dot(x, wr_ref[...], preferred_element_type=jnp.float32)
    _write_eighths(xl, xr, xle_ref, xre_ref)


def _tc_lin2_body(h_ref, wl_ref, wr_ref, xle_ref, xre_ref):
    xl = sum(jnp.dot(h_ref[g], wl_ref[g * _C:(g + 1) * _C, :],
                     preferred_element_type=jnp.float32) for g in range(_H))
    xr = sum(jnp.dot(h_ref[g], wr_ref[g * _C:(g + 1) * _C, :],
                     preferred_element_type=jnp.float32) for g in range(_H))
    _write_eighths(xl, xr, xle_ref, xre_ref)


def _lin_specs(first_in_spec, wshape):
    grid = (_NP // _NB,)
    in_specs = [
        first_in_spec,
        pl.BlockSpec(wshape, lambda i: (0, 0)),
        pl.BlockSpec(wshape, lambda i: (0, 0)),
    ]
    out_specs = [
        pl.BlockSpec((_H, _NB, _C), lambda i: (0, i, 0)),
        pl.BlockSpec((_H, _NB, _C), lambda i: (0, i, 0)),
    ]
    out_shape = [jax.ShapeDtypeStruct((_H, _NP, _C), jnp.float32),
                 jax.ShapeDtypeStruct((_H, _NP, _C), jnp.float32)]
    return dict(grid=grid, in_specs=in_specs, out_specs=out_specs,
                out_shape=out_shape)


def _tc_head_body(h_ref, batch_ref, u_ref, wlin_ref, blin_ref,
                  wo1_ref, wo2_ref, out_ref):
    b = batch_ref[...]
    gids = lax.broadcasted_iota(jnp.int32, (_NP, _G), 1)
    oh = jnp.where(b == gids, 1.0, 0.0).astype(jnp.float32)
    cnt = jnp.sum(oh, axis=0)[:, None]
    pooled = jnp.concatenate(
        [lax.dot_general(oh, h_ref[g], (((0,), (0,)), ((), ())),
                         preferred_element_type=jnp.float32)
         for g in range(_H)], axis=1)
    pooled = pooled / jnp.maximum(cnt, 1.0)
    hr = jnp.maximum(
        jnp.dot(pooled, wlin_ref[...], preferred_element_type=jnp.float32)
        + blin_ref[...][None, :], 0.0)
    out = (jnp.dot(hr, wo1_ref[...], preferred_element_type=jnp.float32)
           + jnp.dot(u_ref[...], wo2_ref[...],
                     preferred_element_type=jnp.float32))
    out_ref[...] = out


def kernel(x, edge_index, batch, u, Wl1, Wr1, att1, b1, Wl2, Wr2, att2, b2,
           W_lin1, b_lin1, W_out, b_out):
    f32 = jnp.float32
    loop = jnp.arange(_N, dtype=edge_index.dtype)
    pad_e = jnp.full((_EP - _E - _N,), _N, edge_index.dtype)
    srcp = jnp.concatenate([edge_index[0], loop, pad_e])
    dstp = jnp.concatenate([edge_index[1], loop, pad_e])

    x_pad = jnp.pad(x, ((0, _NP - _N), (0, 5)))
    wl1p = jnp.pad(Wl1, ((0, 5), (0, 0)))
    wr1p = jnp.pad(Wr1, ((0, 5), (0, 0)))
    att1f = att1.reshape(_H, _C)
    att2f = att2.reshape(_H, _C)
    b1_2d = b1.reshape(2, _FH)
    b2_2d = b2.reshape(2, _FH)
    batch2d = jnp.concatenate(
        [batch, jnp.full((_NP - _N,), _G, batch.dtype)])[:, None]
    u_pad = jnp.pad(u, ((0, 0), (0, 5)))
    wlinp = jnp.pad(W_lin1, ((0, 0), (0, _FH - 64)))
    blinp = jnp.pad(b_lin1, (0, _FH - 64))
    wo1p = jnp.pad(W_out[:64], ((0, _FH - 64), (0, _FH - 1)))
    wo2p = jnp.pad(W_out[64:67], ((0, 5), (0, _FH - 1)))

    xl1e, xr1e = pl.pallas_call(
        _tc_lin1_body,
        **_lin_specs(pl.BlockSpec((_NB, 8), lambda i: (i, 0)), (8, _F)),
    )(x_pad, wl1p, wr1p)
    h1e = _sc_gat(xl1e.reshape(_H * _NP, _C),
                  xr1e.reshape(_H * _NP, _C),
                  srcp, dstp, att1f, b1_2d)

    xl2e, xr2e = pl.pallas_call(
        _tc_lin2_body,
        **_lin_specs(pl.BlockSpec((_H, _NB, _C), lambda i: (0, i, 0)),
                     (_F, _F)),
    )(h1e.reshape(_H, _NP, _C), Wl2, Wr2)
    h2e = _sc_gat(xl2e.reshape(_H * _NP, _C),
                  xr2e.reshape(_H * _NP, _C),
                  srcp, dstp, att2f, b2_2d)

    out = pl.pallas_call(
        _tc_head_body,
        out_shape=jax.ShapeDtypeStruct((_G, _FH), f32),
    )(h2e.reshape(_H, _NP, _C), batch2d, u_pad, wlinp, blinp, wo1p, wo2p)
    return out[:, 0] + b_out[0]


# speedup vs baseline: 12.3526x; 1.3077x over previous
import jax
import jax.numpy as jnp
from jax import lax
from jax.experimental import pallas as pl
from jax.experimental.pallas import tpu as pltpu
from jax.experimental.pallas import tpu_sc as plsc

_N = 10000
_E = 320000
_G = 32
_H = 8
_C = 32
_F = 256
_FH = 128
_HL = 4

_NP = 10240
_EP = 331776
_K = 128
_EPT = _EP // 16
_NCH = _EPT // _K
_RT = _NP // 16
_RH = _RT // 2

_mesh = plsc.VectorSubcoreMesh(
    core_axis_name="c", subcore_axis_name="s", num_cores=2, num_subcores=16
)

_sc_cp = pltpu.CompilerParams(
    needs_layout_passes=False, use_tc_tiling_on_sc=False)


def _sc_gat_body(xle, xre, srch, dsth, atth, bh, ho,
                 dentab, rxl0, rxl1, rxr0, rxr1, srct, dstt,
                 idxa0, idxa1, idxb0, idxb1,
                 evb, attv, bv, fin, dentr, tmpa, idx1,
                 outacc, densh, sem0, sem1):
    cid = lax.axis_index("c")
    sid = lax.axis_index("s")
    base_t = sid * _EPT
    r0 = sid * _RT
    lanes = lax.iota(jnp.int32, 16)
    zero16 = jnp.zeros((16,), jnp.float32)
    zsp = jnp.zeros((16,), jnp.int32)
    hoff = [jnp.full((16,), (4 * cid + h) * _NP, jnp.int32)
            for h in range(_HL)]
    rxl = [rxl0, rxl1]
    rxr = [rxr0, rxr1]
    idxa = [idxa0, idxa1]
    idxb = [idxb0, idxb1]
    sems = [sem0, sem1]

    pltpu.sync_copy(atth.at[pl.ds(cid * _HL, _HL)], attv)
    pltpu.sync_copy(bh.at[cid], bv)
    pltpu.sync_copy(srch.at[pl.ds(base_t, _EPT)], srct)
    pltpu.sync_copy(dsth.at[pl.ds(base_t, _EPT)], dstt)
    att_vecs = [[attv[h, pl.ds(0, 16)], attv[h, pl.ds(16, 16)]]
                for h in range(_HL)]

    def _issue(nci, b, h):
        gb = nci * _K
        for j in range(_K // 16):
            idxa[b][pl.ds(j * 16, 16)] = srct[pl.ds(gb + j * 16, 16)] + hoff[h]
            idxb[b][pl.ds(j * 16, 16)] = dstt[pl.ds(gb + j * 16, 16)] + hoff[h]
        pltpu.async_copy(xle.at[idxa[b]], rxl[b], sems[b])
        pltpu.async_copy(xre.at[idxb[b]], rxr[b], sems[b])

    def _drain(b):
        pltpu.make_async_copy(xle.at[idxa[b]], rxl[b], sems[b]).wait()
        pltpu.make_async_copy(xre.at[idxb[b]], rxr[b], sems[b]).wait()

    for h in range(_HL):
        @pl.loop(0, _NP, step=16)
        def _(j):
            dentab[0, pl.ds(j, 16)] = zero16

        @pl.loop(0, _RT, step=16)
        def _(j):
            tmpa[0, pl.ds(j, 16)] = zero16
        pltpu.sync_copy(tmpa, densh.at[:, pl.ds(r0, _RT)])

        @pl.loop(0, _RH)
        def _(r):
            fin[r, pl.ds(0, 16)] = zero16
            fin[r, pl.ds(16, 16)] = zero16
        pltpu.sync_copy(fin, outacc.at[pl.ds(r0, _RH)])
        pltpu.sync_copy(fin, outacc.at[pl.ds(r0 + _RH, _RH)])
        plsc.subcore_barrier()

        def _compute(cib, b, h):
            gb = cib * _K
            rl = rxl[b]
            rr = rxr[b]

            @pl.loop(0, _K, step=16)
            def _(e0, h=h, rl=rl, rr=rr, gb=gb):
                elanes = lanes + e0
                acc = jnp.zeros((16,), jnp.float32)
                for c in range(_C):
                    csp = jnp.full((16,), c, jnp.int32)
                    v = (plsc.load_gather(rl, [elanes, csp])
                         + plsc.load_gather(rr, [elanes, csp]))
                    m = jnp.maximum(v, v * 0.2)
                    acc = acc + m * att_vecs[h][c // 16][c % 16]
                ev = jnp.exp(acc)
                evb[0, pl.ds(e0, 16)] = ev
                plsc.addupdate_scatter(
                    dentab, [zsp, dstt[pl.ds(gb + e0, 16)]], ev)

            @pl.loop(0, _K)
            def _(e, rl=rl):
                esp = jnp.full((16,), e, jnp.int32)
                a = plsc.load_gather(evb, [zsp, esp])
                rl[e, pl.ds(0, 16)] = rl[e, pl.ds(0, 16)] * a
                rl[e, pl.ds(16, 16)] = rl[e, pl.ds(16, 16)] * a

            pltpu.sync_copy(rl, outacc.at[dstt.at[pl.ds(gb, _K)]], add=True)

        _issue(jnp.int32(0), 0, h)

        @pl.loop(0, _NCH, step=2)
        def _(ci, h=h):
            for b in range(2):
                cib = ci + b
                _issue(jnp.minimum(cib + 1, _NCH - 1), 1 - b, h)
                _drain(b)
                _compute(cib, b, h)

        _drain(0)

        plsc.subcore_barrier()
        plsc.store_scatter(idx1, [lanes],
                           jnp.zeros((16,), jnp.int32), mask=lanes < 1)
        pltpu.sync_copy(dentab, densh.at[idx1], add=True)
        plsc.subcore_barrier()

        pltpu.sync_copy(densh.at[:, pl.ds(r0, _RT)], dentr)
        @pl.loop(0, _RT, step=16)
        def _(j):
            dentr[0, pl.ds(j, 16)] = 1.0 / (dentr[0, pl.ds(j, 16)] + 1e-30)

        for half in range(2):
            pltpu.sync_copy(outacc.at[pl.ds(r0 + half * _RH, _RH)], fin)

            @pl.loop(0, _RH)
            def _(r, h=h, half=half):
                rsp = jnp.full((16,), r + half * _RH, jnp.int32)
                a = plsc.load_gather(dentr, [zsp, rsp])
                for j in range(_C // 16):
                    v = (fin[r, pl.ds(j * 16, 16)] * a
                         + bv[pl.ds(h * _C + j * 16, 16)])
                    fin[r, pl.ds(j * 16, 16)] = jnp.maximum(v, 0.0)
            pltpu.sync_copy(
                fin,
                ho.at[pl.ds((4 * cid + h) * _NP + r0 + half * _RH, _RH)])
        plsc.subcore_barrier()


_sc_gat = pl.kernel(
    _sc_gat_body,
    out_type=jax.ShapeDtypeStruct((_H * _NP, _C), jnp.float32),
    mesh=_mesh,
    scratch_types=[
        pltpu.VMEM((1, _NP), jnp.float32),
        pltpu.VMEM((_K, _C), jnp.float32),
        pltpu.VMEM((_K, _C), jnp.float32),
        pltpu.VMEM((_K, _C), jnp.float32),
        pltpu.VMEM((_K, _C), jnp.float32),
        pltpu.VMEM((_EPT,), jnp.int32),
        pltpu.VMEM((_EPT,), jnp.int32),
        pltpu.VMEM((_K,), jnp.int32),
        pltpu.VMEM((_K,), jnp.int32),
        pltpu.VMEM((_K,), jnp.int32),
        pltpu.VMEM((_K,), jnp.int32),
        pltpu.VMEM((1, _K), jnp.float32),
        pltpu.VMEM((_HL, _C), jnp.float32),
        pltpu.VMEM((_FH,), jnp.float32),
        pltpu.VMEM((_RH, _C), jnp.float32),
        pltpu.VMEM((1, _RT), jnp.float32),
        pltpu.VMEM((1, _RT), jnp.float32),
        pltpu.VMEM((1,), jnp.int32),
        pltpu.VMEM_SHARED((_NP, _C), jnp.float32),
        pltpu.VMEM_SHARED((1, _NP), jnp.float32),
        pltpu.SemaphoreType.DMA,
        pltpu.SemaphoreType.DMA,
    ],
    compiler_params=_sc_cp,
)


_NB = 2048
_FQ = 64


def _write_eighths(xl, xr, xle_ref, xre_ref):
    for g in range(_H):
        xle_ref[g] = xl[:, g * _C:(g + 1) * _C]
        xre_ref[g] = xr[:, g * _C:(g + 1) * _C]


def _tc_lin1_body(x_ref, wl_ref, wr_ref, xle_ref, xre_ref):
    x = x_ref[...]
    xl = jnp.dot(x, wl_ref[...], preferred_element_type=jnp.float32)
    xr = jnp.dot(x, wr_ref[...], preferred_element_type=jnp.float32)
    _write_eighths(xl, xr, xle_ref, xre_ref)


def _tc_lin2_body(h_ref, wl_ref, wr_ref, xle_ref, xre_ref):
    xl = sum(jnp.dot(h_ref[g], wl_ref[g * _C:(g + 1) * _C, :],
                     preferred_element_type=jnp.float32) for g in range(_H))
    xr = sum(jnp.dot(h_ref[g], wr_ref[g * _C:(g + 1) * _C, :],
                     preferred_element_type=jnp.float32) for g in range(_H))
    _write_eighths(xl, xr, xle_ref, xre_ref)


def _lin_specs(first_in_spec, wshape):
    grid = (_NP // _NB,)
    in_specs = [
        first_in_spec,
        pl.BlockSpec(wshape, lambda i: (0, 0)),
        pl.BlockSpec(wshape, lambda i: (0, 0)),
    ]
    out_specs = [
        pl.BlockSpec((_H, _NB, _C), lambda i: (0, i, 0)),
        pl.BlockSpec((_H, _NB, _C), lambda i: (0, i, 0)),
    ]
    out_shape = [jax.ShapeDtypeStruct((_H, _NP, _C), jnp.float32),
                 jax.ShapeDtypeStruct((_H, _NP, _C), jnp.float32)]
    return dict(grid=grid, in_specs=in_specs, out_specs=out_specs,
                out_shape=out_shape)


def _tc_head_body(h_ref, batch_ref, u_ref, wlin_ref, blin_ref,
                  wo1_ref, wo2_ref, out_ref):
    b = batch_ref[...]
    gids = lax.broadcasted_iota(jnp.int32, (_NP, _G), 1)
    oh = jnp.where(b == gids, 1.0, 0.0).astype(jnp.float32)
    cnt = jnp.sum(oh, axis=0)[:, None]
    pooled = jnp.concatenate(
        [lax.dot_general(oh, h_ref[g], (((0,), (0,)), ((), ())),
                         preferred_element_type=jnp.float32)
         for g in range(_H)], axis=1)
    pooled = pooled / jnp.maximum(cnt, 1.0)
    hr = jnp.maximum(
        jnp.dot(pooled, wlin_ref[...], preferred_element_type=jnp.float32)
        + blin_ref[...][None, :], 0.0)
    out = (jnp.dot(hr, wo1_ref[...], preferred_element_type=jnp.float32)
           + jnp.dot(u_ref[...], wo2_ref[...],
                     preferred_element_type=jnp.float32))
    out_ref[...] = out


def kernel(x, edge_index, batch, u, Wl1, Wr1, att1, b1, Wl2, Wr2, att2, b2,
           W_lin1, b_lin1, W_out, b_out):
    f32 = jnp.float32
    loop = jnp.arange(_N, dtype=edge_index.dtype)
    pad_e = jnp.full((_EP - _E - _N,), _N, edge_index.dtype)
    srcp = jnp.concatenate([edge_index[0], loop, pad_e])
    dstp = jnp.concatenate([edge_index[1], loop, pad_e])

    x_pad = jnp.pad(x, ((0, _NP - _N), (0, 5)))
    wl1p = jnp.pad(Wl1, ((0, 5), (0, 0)))
    wr1p = jnp.pad(Wr1, ((0, 5), (0, 0)))
    att1f = att1.reshape(_H, _C)
    att2f = att2.reshape(_H, _C)
    b1_2d = b1.reshape(2, _FH)
    b2_2d = b2.reshape(2, _FH)
    batch2d = jnp.concatenate(
        [batch, jnp.full((_NP - _N,), _G, batch.dtype)])[:, None]
    u_pad = jnp.pad(u, ((0, 0), (0, 5)))
    wlinp = jnp.pad(W_lin1, ((0, 0), (0, _FH - 64)))
    blinp = jnp.pad(b_lin1, (0, _FH - 64))
    wo1p = jnp.pad(W_out[:64], ((0, _FH - 64), (0, _FH - 1)))
    wo2p = jnp.pad(W_out[64:67], ((0, 5), (0, _FH - 1)))

    xl1e, xr1e = pl.pallas_call(
        _tc_lin1_body,
        **_lin_specs(pl.BlockSpec((_NB, 8), lambda i: (i, 0)), (8, _F)),
    )(x_pad, wl1p, wr1p)
    h1e = _sc_gat(xl1e.reshape(_H * _NP, _C),
                  xr1e.reshape(_H * _NP, _C),
                  srcp, dstp, att1f, b1_2d)

    xl2e, xr2e = pl.pallas_call(
        _tc_lin2_body,
        **_lin_specs(pl.BlockSpec((_H, _NB, _C), lambda i: (0, i, 0)),
                     (_F, _F)),
    )(h1e.reshape(_H, _NP, _C), Wl2, Wr2)
    h2e = _sc_gat(xl2e.reshape(_H * _NP, _C),
                  xr2e.reshape(_H * _NP, _C),
                  srcp, dstp, att2f, b2_2d)

    out = pl.pallas_call(
        _tc_head_body,
        out_shape=jax.ShapeDtypeStruct((_G, _FH), f32),
    )(h2e.reshape(_H, _NP, _C), batch2d, u_pad, wlinp, blinp, wo1p, wo2p)
    return out[:, 0] + b_out[0]


# speedup vs baseline: 12.3982x; 1.0037x over previous
import jax
import jax.numpy as jnp
from jax import lax
from jax.experimental import pallas as pl
from jax.experimental.pallas import tpu as pltpu
from jax.experimental.pallas import tpu_sc as plsc

_N = 10000
_E = 320000
_G = 32
_H = 8
_C = 32
_F = 256
_FH = 128
_HL = 4

_NP = 10240
_EP = 331776
_K = 192
_EPT = _EP // 16
_NCH = _EPT // _K
_RT = _NP // 16
_RH = _RT // 2

_mesh = plsc.VectorSubcoreMesh(
    core_axis_name="c", subcore_axis_name="s", num_cores=2, num_subcores=16
)

_sc_cp = pltpu.CompilerParams(
    needs_layout_passes=False, use_tc_tiling_on_sc=False)


def _sc_gat_body(xle, xre, srch, dsth, atth, bh, ho,
                 dentab, rxl0, rxl1, rxr0, rxr1, srct, dstt,
                 idxa0, idxa1, idxb0, idxb1,
                 evb, attv, bv, fin, dentr, tmpa, idx1,
                 outacc, densh, sem0, sem1):
    cid = lax.axis_index("c")
    sid = lax.axis_index("s")
    base_t = sid * _EPT
    r0 = sid * _RT
    lanes = lax.iota(jnp.int32, 16)
    zero16 = jnp.zeros((16,), jnp.float32)
    zsp = jnp.zeros((16,), jnp.int32)
    hoff = [jnp.full((16,), (4 * cid + h) * _NP, jnp.int32)
            for h in range(_HL)]
    rxl = [rxl0, rxl1]
    rxr = [rxr0, rxr1]
    idxa = [idxa0, idxa1]
    idxb = [idxb0, idxb1]
    sems = [sem0, sem1]

    pltpu.sync_copy(atth.at[pl.ds(cid * _HL, _HL)], attv)
    pltpu.sync_copy(bh.at[cid], bv)
    pltpu.sync_copy(srch.at[pl.ds(base_t, _EPT)], srct)
    pltpu.sync_copy(dsth.at[pl.ds(base_t, _EPT)], dstt)
    att_vecs = [[attv[h, pl.ds(0, 16)], attv[h, pl.ds(16, 16)]]
                for h in range(_HL)]

    def _issue(nci, b, h):
        gb = nci * _K
        for j in range(_K // 16):
            idxa[b][pl.ds(j * 16, 16)] = srct[pl.ds(gb + j * 16, 16)] + hoff[h]
            idxb[b][pl.ds(j * 16, 16)] = dstt[pl.ds(gb + j * 16, 16)] + hoff[h]
        pltpu.async_copy(xle.at[idxa[b]], rxl[b], sems[b])
        pltpu.async_copy(xre.at[idxb[b]], rxr[b], sems[b])

    def _drain(b):
        pltpu.make_async_copy(xle.at[idxa[b]], rxl[b], sems[b]).wait()
        pltpu.make_async_copy(xre.at[idxb[b]], rxr[b], sems[b]).wait()

    for h in range(_HL):
        @pl.loop(0, _NP, step=16)
        def _(j):
            dentab[0, pl.ds(j, 16)] = zero16

        @pl.loop(0, _RT, step=16)
        def _(j):
            tmpa[0, pl.ds(j, 16)] = zero16
        pltpu.sync_copy(tmpa, densh.at[:, pl.ds(r0, _RT)])

        @pl.loop(0, _RH)
        def _(r):
            fin[r, pl.ds(0, 16)] = zero16
            fin[r, pl.ds(16, 16)] = zero16
        pltpu.sync_copy(fin, outacc.at[pl.ds(r0, _RH)])
        pltpu.sync_copy(fin, outacc.at[pl.ds(r0 + _RH, _RH)])
        plsc.subcore_barrier()

        def _compute(cib, b, h):
            gb = cib * _K
            rl = rxl[b]
            rr = rxr[b]

            @pl.loop(0, _K, step=16)
            def _(e0, h=h, rl=rl, rr=rr, gb=gb):
                elanes = lanes + e0
                acc = jnp.zeros((16,), jnp.float32)
                for c in range(_C):
                    csp = jnp.full((16,), c, jnp.int32)
                    v = (plsc.load_gather(rl, [elanes, csp])
                         + plsc.load_gather(rr, [elanes, csp]))
                    m = jnp.maximum(v, v * 0.2)
                    acc = acc + m * att_vecs[h][c // 16][c % 16]
                ev = jnp.exp(acc)
                evb[0, pl.ds(e0, 16)] = ev
                plsc.addupdate_scatter(
                    dentab, [zsp, dstt[pl.ds(gb + e0, 16)]], ev)

            @pl.loop(0, _K)
            def _(e, rl=rl):
                esp = jnp.full((16,), e, jnp.int32)
                a = plsc.load_gather(evb, [zsp, esp])
                rl[e, pl.ds(0, 16)] = rl[e, pl.ds(0, 16)] * a
                rl[e, pl.ds(16, 16)] = rl[e, pl.ds(16, 16)] * a

            pltpu.sync_copy(rl, outacc.at[dstt.at[pl.ds(gb, _K)]], add=True)

        _issue(jnp.int32(0), 0, h)

        @pl.loop(0, _NCH, step=2)
        def _(ci, h=h):
            for b in range(2):
                cib = ci + b
                _issue(jnp.minimum(cib + 1, _NCH - 1), 1 - b, h)
                _drain(b)
                _compute(cib, b, h)

        _drain(0)

        plsc.subcore_barrier()
        plsc.store_scatter(idx1, [lanes],
                           jnp.zeros((16,), jnp.int32), mask=lanes < 1)
        pltpu.sync_copy(dentab, densh.at[idx1], add=True)
        plsc.subcore_barrier()

        pltpu.sync_copy(densh.at[:, pl.ds(r0, _RT)], dentr)
        @pl.loop(0, _RT, step=16)
        def _(j):
            dentr[0, pl.ds(j, 16)] = 1.0 / (dentr[0, pl.ds(j, 16)] + 1e-30)

        for half in range(2):
            pltpu.sync_copy(outacc.at[pl.ds(r0 + half * _RH, _RH)], fin)

            @pl.loop(0, _RH)
            def _(r, h=h, half=half):
                rsp = jnp.full((16,), r + half * _RH, jnp.int32)
                a = plsc.load_gather(dentr, [zsp, rsp])
                for j in range(_C // 16):
                    v = (fin[r, pl.ds(j * 16, 16)] * a
                         + bv[pl.ds(h * _C + j * 16, 16)])
                    fin[r, pl.ds(j * 16, 16)] = jnp.maximum(v, 0.0)
            pltpu.sync_copy(
                fin,
                ho.at[pl.ds((4 * cid + h) * _NP + r0 + half * _RH, _RH)])
        plsc.subcore_barrier()


_sc_gat = pl.kernel(
    _sc_gat_body,
    out_type=jax.ShapeDtypeStruct((_H * _NP, _C), jnp.float32),
    mesh=_mesh,
    scratch_types=[
        pltpu.VMEM((1, _NP), jnp.float32),
        pltpu.VMEM((_K, _C), jnp.float32),
        pltpu.VMEM((_K, _C), jnp.float32),
        pltpu.VMEM((_K, _C), jnp.float32),
        pltpu.VMEM((_K, _C), jnp.float32),
        pltpu.VMEM((_EPT,), jnp.int32),
        pltpu.VMEM((_EPT,), jnp.int32),
        pltpu.VMEM((_K,), jnp.int32),
        pltpu.VMEM((_K,), jnp.int32),
        pltpu.VMEM((_K,), jnp.int32),
        pltpu.VMEM((_K,), jnp.int32),
        pltpu.VMEM((1, _K), jnp.float32),
        pltpu.VMEM((_HL, _C), jnp.float32),
        pltpu.VMEM((_FH,), jnp.float32),
        pltpu.VMEM((_RH, _C), jnp.float32),
        pltpu.VMEM((1, _RT), jnp.float32),
        pltpu.VMEM((1, _RT), jnp.float32),
        pltpu.VMEM((1,), jnp.int32),
        pltpu.VMEM_SHARED((_NP, _C), jnp.float32),
        pltpu.VMEM_SHARED((1, _NP), jnp.float32),
        pltpu.SemaphoreType.DMA,
        pltpu.SemaphoreType.DMA,
    ],
    compiler_params=_sc_cp,
)


_NB = 2048
_FQ = 64


def _write_eighths(xl, xr, xle_ref, xre_ref):
    for g in range(_H):
        xle_ref[g] = xl[:, g * _C:(g + 1) * _C]
        xre_ref[g] = xr[:, g * _C:(g + 1) * _C]


def _tc_lin1_body(x_ref, wl_ref, wr_ref, xle_ref, xre_ref):
    x = x_ref[...]
    xl = jnp.dot(x, wl_ref[...], preferred_element_type=jnp.float32)
    xr = jnp.dot(x, wr_ref[...], preferred_element_type=jnp.float32)
    _write_eighths(xl, xr, xle_ref, xre_ref)


def _tc_lin2_body(h_ref, wl_ref, wr_ref, xle_ref, xre_ref):
    xl = sum(jnp.dot(h_ref[g], wl_ref[g * _C:(g + 1) * _C, :],
                     preferred_element_type=jnp.float32) for g in range(_H))
    xr = sum(jnp.dot(h_ref[g], wr_ref[g * _C:(g + 1) * _C, :],
                     preferred_element_type=jnp.float32) for g in range(_H))
    _write_eighths(xl, xr, xle_ref, xre_ref)


def _lin_specs(first_in_spec, wshape):
    grid = (_NP // _NB,)
    in_specs = [
        first_in_spec,
        pl.BlockSpec(wshape, lambda i: (0, 0)),
        pl.BlockSpec(wshape, lambda i: (0, 0)),
    ]
    out_specs = [
        pl.BlockSpec((_H, _NB, _C), lambda i: (0, i, 0)),
        pl.BlockSpec((_H, _NB, _C), lambda i: (0, i, 0)),
    ]
    out_shape = [jax.ShapeDtypeStruct((_H, _NP, _C), jnp.float32),
                 jax.ShapeDtypeStruct((_H, _NP, _C), jnp.float32)]
    return dict(grid=grid, in_specs=in_specs, out_specs=out_specs,
                out_shape=out_shape)


def _tc_head_body(h_ref, batch_ref, u_ref, wlin_ref, blin_ref,
                  wo1_ref, wo2_ref, out_ref):
    b = batch_ref[...]
    gids = lax.broadcasted_iota(jnp.int32, (_NP, _G), 1)
    oh = jnp.where(b == gids, 1.0, 0.0).astype(jnp.float32)
    cnt = jnp.sum(oh, axis=0)[:, None]
    pooled = jnp.concatenate(
        [lax.dot_general(oh, h_ref[g], (((0,), (0,)), ((), ())),
                         preferred_element_type=jnp.float32)
         for g in range(_H)], axis=1)
    pooled = pooled / jnp.maximum(cnt, 1.0)
    hr = jnp.maximum(
        jnp.dot(pooled, wlin_ref[...], preferred_element_type=jnp.float32)
        + blin_ref[...][None, :], 0.0)
    out = (jnp.dot(hr, wo1_ref[...], preferred_element_type=jnp.float32)
           + jnp.dot(u_ref[...], wo2_ref[...],
                     preferred_element_type=jnp.float32))
    out_ref[...] = out


def kernel(x, edge_index, batch, u, Wl1, Wr1, att1, b1, Wl2, Wr2, att2, b2,
           W_lin1, b_lin1, W_out, b_out):
    f32 = jnp.float32
    loop = jnp.arange(_N, dtype=edge_index.dtype)
    pad_e = jnp.full((_EP - _E - _N,), _N, edge_index.dtype)
    srcp = jnp.concatenate([edge_index[0], loop, pad_e])
    dstp = jnp.concatenate([edge_index[1], loop, pad_e])

    x_pad = jnp.pad(x, ((0, _NP - _N), (0, 5)))
    wl1p = jnp.pad(Wl1, ((0, 5), (0, 0)))
    wr1p = jnp.pad(Wr1, ((0, 5), (0, 0)))
    att1f = att1.reshape(_H, _C)
    att2f = att2.reshape(_H, _C)
    b1_2d = b1.reshape(2, _FH)
    b2_2d = b2.reshape(2, _FH)
    batch2d = jnp.concatenate(
        [batch, jnp.full((_NP - _N,), _G, batch.dtype)])[:, None]
    u_pad = jnp.pad(u, ((0, 0), (0, 5)))
    wlinp = jnp.pad(W_lin1, ((0, 0), (0, _FH - 64)))
    blinp = jnp.pad(b_lin1, (0, _FH - 64))
    wo1p = jnp.pad(W_out[:64], ((0, _FH - 64), (0, _FH - 1)))
    wo2p = jnp.pad(W_out[64:67], ((0, 5), (0, _FH - 1)))

    xl1e, xr1e = pl.pallas_call(
        _tc_lin1_body,
        **_lin_specs(pl.BlockSpec((_NB, 8), lambda i: (i, 0)), (8, _F)),
    )(x_pad, wl1p, wr1p)
    h1e = _sc_gat(xl1e.reshape(_H * _NP, _C),
                  xr1e.reshape(_H * _NP, _C),
                  srcp, dstp, att1f, b1_2d)

    xl2e, xr2e = pl.pallas_call(
        _tc_lin2_body,
        **_lin_specs(pl.BlockSpec((_H, _NB, _C), lambda i: (0, i, 0)),
                     (_F, _F)),
    )(h1e.reshape(_H, _NP, _C), Wl2, Wr2)
    h2e = _sc_gat(xl2e.reshape(_H * _NP, _C),
                  xr2e.reshape(_H * _NP, _C),
                  srcp, dstp, att2f, b2_2d)

    out = pl.pallas_call(
        _tc_head_body,
        out_shape=jax.ShapeDtypeStruct((_G, _FH), f32),
    )(h2e.reshape(_H, _NP, _C), batch2d, u_pad, wlinp, blinp, wo1p, wo2p)
    return out[:, 0] + b_out[0]
